# baseline (speedup 1.0000x reference)
import functools

import jax
import jax.numpy as jnp
import numpy as np
from jax import lax
from jax.experimental import pallas as pl
from jax.experimental.pallas import tpu as pltpu
from jax.experimental.pallas import tpu_sc as plsc

F_ = 26
V = 100000
K = 16
DFF = 128
NLAYERS = 5

_NC = 2
_NS = 16
_NW = _NC * _NS
_CH = 128


def _sc_gather(tab2, tab1, idx3d, n_per, n_ch):
    N = _NW * n_per
    mesh = plsc.VectorSubcoreMesh(core_axis_name="c", subcore_axis_name="s")

    @functools.partial(
        pl.kernel,
        mesh=mesh,
        compiler_params=pltpu.CompilerParams(use_tc_tiling_on_sc=False),
        out_type=(
            jax.ShapeDtypeStruct((N, K), jnp.float32),
            jax.ShapeDtypeStruct((N, 1), jnp.float32),
        ),
        scratch_types=[
            pltpu.VMEM((n_ch, _CH), jnp.int32),
            pltpu.VMEM((n_per, K), jnp.float32),
            pltpu.VMEM((n_per, 1), jnp.float32),
            pltpu.SemaphoreType.DMA,
            pltpu.SemaphoreType.DMA,
        ],
    )
    def gather_kernel(idx_hbm, tab2_hbm, tab1_hbm, out2_hbm, out1_hbm,
                      idx_v, rows_v, w1_v, sem2, sem1):
        wid = lax.axis_index("s") * _NC + lax.axis_index("c")
        pltpu.sync_copy(idx_hbm.at[wid], idx_v)

        def body(c, carry):
            pltpu.async_copy(
                tab2_hbm.at[idx_v.at[c]],
                rows_v.at[pl.ds(c * _CH, _CH)], sem2).wait()
            pltpu.async_copy(
                tab1_hbm.at[idx_v.at[c]],
                w1_v.at[pl.ds(c * _CH, _CH)], sem1).wait()
            return carry

        lax.fori_loop(0, n_ch, body, 0)
        base = wid * n_per
        pltpu.sync_copy(rows_v, out2_hbm.at[pl.ds(base, n_per)])
        pltpu.sync_copy(w1_v, out1_hbm.at[pl.ds(base, n_per)])

    return gather_kernel(idx3d, tab2, tab1)


def _norm(x, a, b, eps=1e-6):
    m = x.mean(axis=-1, keepdims=True)
    sd = jnp.std(x, axis=-1, keepdims=True, ddof=1)
    return a * (x - m) / (sd + eps) + b


def _bn_eval(x, eps=1e-5):
    return x / jnp.sqrt(1.0 + eps)


def _encoder(x, p):
    x2 = _norm(x, p['n1_a'], p['n1_b'])
    q = x2 @ p['wq'].T + p['bq']
    k = x2 @ p['wk'].T + p['bk']
    v = x2 @ p['wv'].T + p['bv']
    scores = jnp.einsum('bfd,bgd->bfg', q, k) / np.sqrt(K)
    att = jnp.einsum('bfg,bgd->bfd', scores, v)
    att = att @ p['wo'].T + p['bo']
    x = x + att
    x2 = _norm(x, p['n2_a'], p['n2_b'])
    h = x2 @ p['ffw1'].T + p['ffb1']
    h = jax.nn.relu(_bn_eval(h))
    ff = h @ p['ffw2'].T + p['ffb2']
    return x + ff


def kernel(Xi, Xv, params, pe):
    B = Xi.shape[0]
    N = B * F_

    idx = Xi[..., 0].astype(jnp.int32) + (jnp.arange(F_, dtype=jnp.int32) * V)[None, :]
    n_per = N // _NW
    n_ch = n_per // _CH
    idx3d = idx.reshape(_NW, n_ch, _CH)
    tab2 = params['fm_w2'].reshape(F_ * V, K)
    tab1 = params['fm_w1'].reshape(F_ * V, 1)
    rows2, rows1 = _sc_gather(tab2, tab1, idx3d, n_per, n_ch)

    w1 = rows1.reshape(B, F_)
    w2r = rows2.reshape(B, F_, K)

    first = w1 * Xv
    w2 = w2r * Xv[..., None]
    ssum = w2.sum(axis=1)
    second = 0.5 * (ssum * ssum - (w2 * w2).sum(axis=1))
    x = w2 * np.sqrt(K) + pe[None, :, :]
    for p in params['enc']:
        x = _encoder(x, p)
    x = _norm(x, params['norm2_a'], params['norm2_b'])
    deep = x.reshape(x.shape[0], -1)
    m0 = first @ params['m0_w'].T + params['m0_b']
    m1 = second @ params['m1_w'].T + params['m1_b']
    m2 = deep @ params['m2_w'].T + params['m2_b']
    cat = jnp.concatenate([m0, m1, m2], axis=1)
    h = cat @ params['cls_w1'].T + params['cls_b1']
    h = jax.nn.relu(_bn_eval(h))
    out = h @ params['cls_w2'].T + params['cls_b2']
    return out
